# Initial kernel scaffold; baseline (speedup 1.0000x reference)
import jax
import jax.numpy as jnp
from jax.experimental import pallas as pl


def kernel(boxes, scores):
    raise NotImplementedError("write your pallas kernel here")


# speedup vs baseline: 15.6480x; 15.6480x over previous
import functools
import jax
import jax.numpy as jnp
from jax import lax
from jax.experimental import pallas as pl
from jax.experimental.pallas import tpu as pltpu

N = 5000
NP = 5120
TILE = 128
T = NP // TILE
IOU_THRESH = 0.3


def _nms_body(bs_ref, bst_ref, keep_ref, iou_ref, supp_ref):
    supp_ref[...] = jnp.zeros((1, NP), jnp.float32)

    lane = lax.broadcasted_iota(jnp.int32, (1, TILE), 1)

    def tile_step(j, _):
        cols = pl.ds(j * TILE, TILE)
        ax1 = bst_ref[cols, 0:1]
        ay1 = bst_ref[cols, 1:2]
        ax2 = bst_ref[cols, 2:3]
        ay2 = bst_ref[cols, 3:4]
        aar = bst_ref[cols, 4:5]
        bx1 = bs_ref[0:1, cols]
        by1 = bs_ref[1:2, cols]
        bx2 = bs_ref[2:3, cols]
        by2 = bs_ref[3:4, cols]
        bar = bs_ref[4:5, cols]

        iw = jnp.clip(jnp.minimum(ax2, bx2) - jnp.maximum(ax1, bx1), 0.0)
        ih = jnp.clip(jnp.minimum(ay2, by2) - jnp.maximum(ay1, by1), 0.0)
        inter = iw * ih
        iou_ref[...] = inter / ((aar + bar - inter) + 1e-9)

        supp_tile = supp_ref[0:1, cols]

        def seq_step(i, st):
            onehot = (lane == i).astype(jnp.float32)
            s_i = jnp.max(st * onehot)
            row = iou_ref[pl.ds(i, 1), :]
            hit = jnp.logical_and(row > IOU_THRESH, lane > i)
            st_new = jnp.maximum(st, hit.astype(jnp.float32))
            return jnp.where(s_i > 0.0, st, st_new)

        supp_tile = lax.fori_loop(0, TILE, seq_step, supp_tile)
        keep_col = 1.0 - supp_tile
        keep_ref[0:1, cols] = keep_col
        kept_row = keep_col.reshape(TILE, 1)

        def cross_step(k, _):
            kcols = pl.ds(k * TILE, TILE)
            cx1 = bs_ref[0:1, kcols]
            cy1 = bs_ref[1:2, kcols]
            cx2 = bs_ref[2:3, kcols]
            cy2 = bs_ref[3:4, kcols]
            car = bs_ref[4:5, kcols]
            iw2 = jnp.clip(jnp.minimum(ax2, cx2) - jnp.maximum(ax1, cx1), 0.0)
            ih2 = jnp.clip(jnp.minimum(ay2, cy2) - jnp.maximum(ay1, cy1), 0.0)
            inter2 = iw2 * ih2
            iou2 = inter2 / ((aar + car - inter2) + 1e-9)
            hit = jnp.max(iou2 * kept_row, axis=0, keepdims=True) > IOU_THRESH
            scol = supp_ref[0:1, kcols]
            supp_ref[0:1, kcols] = jnp.maximum(scol, hit.astype(jnp.float32))
            return 0

        lax.fori_loop(j + 1, T, cross_step, 0)
        return 0

    lax.fori_loop(0, T, tile_step, 0)


@functools.partial(jax.jit)
def _nms_pallas(bs, bst):
    return pl.pallas_call(
        _nms_body,
        out_shape=jax.ShapeDtypeStruct((1, NP), jnp.float32),
        in_specs=[
            pl.BlockSpec(memory_space=pltpu.VMEM),
            pl.BlockSpec(memory_space=pltpu.VMEM),
        ],
        out_specs=pl.BlockSpec(memory_space=pltpu.VMEM),
        scratch_shapes=[
            pltpu.VMEM((TILE, TILE), jnp.float32),
            pltpu.VMEM((1, NP), jnp.float32),
        ],
    )(bs, bst)


def kernel(boxes, scores):
    order = jnp.argsort(-scores)
    b = jnp.take(boxes, order, axis=0)
    area = (b[:, 2] - b[:, 0]) * (b[:, 3] - b[:, 1])
    feats = jnp.concatenate([b, area[:, None]], axis=1)
    featsp = jnp.zeros((NP, 8), jnp.float32).at[:N, :5].set(feats)
    bs = featsp.T
    keep_sorted = _nms_pallas(bs, featsp)[0, :N]
    keep = jnp.zeros((N,), jnp.float32).at[order].set(keep_sorted)
    dets = jnp.concatenate([scores[:, None], boxes], axis=1)
    return dets * keep[:, None]
